# baseline (speedup 1.0000x reference)
import functools

import jax
import jax.numpy as jnp
from jax import lax
from jax.experimental import pallas as pl
from jax.experimental.pallas import tpu as pltpu
from jax.experimental.pallas import tpu_sc as plsc

N = 10000
E = 160000
H = 256
HH = 128
DOUT = 128
L = 3
T = 3

NC = 2
NS = 16
CHUNK = 128

IDXBLK = 16
C16 = 80
PAD16 = NS * C16 * CHUNK - E
C32 = -(-(E // (NC * NS)) // CHUNK)
PAD32 = NC * NS * C32 * CHUNK - E

ROWS_PT = 632
NPAD = NS * ROWS_PT
BLK = 1000
GRID = N // BLK


@functools.partial(
    pl.kernel,
    out_type=jax.ShapeDtypeStruct((T, NC, NPAD, HH), jnp.float32),
    mesh=plsc.VectorSubcoreMesh(core_axis_name="c", subcore_axis_name="s", num_cores=NC, num_subcores=NS),
    scratch_types=[
        pltpu.VMEM((2, IDXBLK, CHUNK), jnp.int32),
        pltpu.VMEM((2, IDXBLK, CHUNK), jnp.int32),
        pltpu.VMEM((2, CHUNK, HH), jnp.float32),
        pltpu.VMEM_SHARED((NPAD, HH), jnp.float32),
        pltpu.SemaphoreType.DMA,
        pltpu.SemaphoreType.DMA,
        pltpu.SemaphoreType.DMA,
    ],
)
def _sc_agg(h_hbm, src_hbm, dst_hbm, z_hbm, dep_hbm, out_hbm, sidx, didx, gbuf,
            aggsh, gsem, ssem, isem):
    c = lax.axis_index("c")
    s = lax.axis_index("s")
    r0 = s * ROWS_PT
    pltpu.sync_copy(z_hbm, aggsh.at[pl.ds(r0, ROWS_PT)])
    plsc.subcore_barrier()

    NMB = C16 // IDXBLK
    blocks = [(t, m) for t in range(T) for m in range(NMB)]

    def _stage_idx(t, m, slot, copy):
        copy(src_hbm.at[t, c, s, pl.ds(m * IDXBLK, IDXBLK)], sidx.at[slot])
        copy(dst_hbm.at[t, s, pl.ds(m * IDXBLK, IDXBLK)], didx.at[slot])

    _stage_idx(0, 0, 0, pltpu.sync_copy)
    pltpu.async_copy(h_hbm.at[sidx.at[0, 0]], gbuf.at[0], gsem)

    for k, (t, m) in enumerate(blocks):
        ks = k % 2
        nks = (k + 1) % 2
        if k + 1 < len(blocks):
            nt, nm = blocks[k + 1]
            _stage_idx(nt, nm, nks,
                       lambda a, b: pltpu.async_copy(a, b, isem))

        def body(j, carry, ks=ks):
            cur = lax.rem(j, 2)
            nxt = lax.rem(j + 1, 2)

            pltpu.make_async_copy(h_hbm.at[sidx.at[ks, j]], gbuf.at[cur],
                                  gsem).wait()
            pltpu.async_copy(gbuf.at[cur], aggsh.at[didx.at[ks, j]], ssem,
                             add=True)

            @pl.when(j >= 1)
            def _():
                pltpu.make_async_copy(z_hbm.at[pl.ds(0, CHUNK)],
                                      gbuf.at[nxt], ssem).wait()

            @pl.when(j + 1 < IDXBLK)
            def _():
                pltpu.async_copy(h_hbm.at[sidx.at[ks, j + 1]], gbuf.at[nxt],
                                 gsem)
            return carry

        lax.fori_loop(0, IDXBLK, body, 0)
        pltpu.make_async_copy(z_hbm.at[pl.ds(0, CHUNK)], gbuf.at[0],
                              ssem).wait()
        if k + 1 < len(blocks):
            nt, nm = blocks[k + 1]
            _stage_idx(nt, nm, nks,
                       lambda a, b: pltpu.make_async_copy(a, b, isem).wait())
            pltpu.async_copy(h_hbm.at[sidx.at[nks, 0]], gbuf.at[0], gsem)

        if m == NMB - 1:
            plsc.subcore_barrier()
            pltpu.sync_copy(aggsh.at[pl.ds(r0, ROWS_PT)],
                            out_hbm.at[t, c, pl.ds(r0, ROWS_PT)])
            if t + 1 < T:
                pltpu.sync_copy(z_hbm, aggsh.at[pl.ds(r0, ROWS_PT)])
                plsc.subcore_barrier()


@functools.partial(
    pl.kernel,
    out_type=jax.ShapeDtypeStruct((T, NC, NPAD, HH), jnp.float32),
    mesh=plsc.VectorSubcoreMesh(core_axis_name="c", subcore_axis_name="s", num_cores=NC, num_subcores=NS),
    scratch_types=[
        pltpu.VMEM((C32, CHUNK), jnp.int32),
        pltpu.VMEM((CHUNK, HH), jnp.float32),
        pltpu.VMEM_SHARED((NPAD, HH), jnp.float32),
        pltpu.SemaphoreType.DMA,
    ],
)
def _sc_count(dst_hbm, ones_hbm, z_hbm, out_hbm, didx, ones, cntsh, ssem):
    c = lax.axis_index("c")
    s = lax.axis_index("s")
    wid = s * NC + c
    r0 = s * ROWS_PT
    pltpu.sync_copy(ones_hbm, ones)
    pltpu.sync_copy(z_hbm, cntsh.at[pl.ds(r0, ROWS_PT)])
    plsc.subcore_barrier()

    for t in range(T):
        pltpu.sync_copy(dst_hbm.at[t, wid], didx)

        def body(j, carry):
            pltpu.async_copy(ones, cntsh.at[didx.at[j]], ssem, add=True)
            return carry

        lax.fori_loop(0, C32, body, 0)

        def drain(j, carry):
            pltpu.make_async_copy(z_hbm.at[pl.ds(0, CHUNK)], ones,
                                  ssem).wait()
            return carry

        lax.fori_loop(0, C32, drain, 0)
        plsc.subcore_barrier()
        pltpu.sync_copy(cntsh.at[pl.ds(r0, ROWS_PT)],
                        out_hbm.at[t, c, pl.ds(r0, ROWS_PT)])
        if t + 1 < T:
            pltpu.sync_copy(z_hbm, cntsh.at[pl.ds(r0, ROWS_PT)])
            plsc.subcore_barrier()


def _dot(a, b):
    return jnp.dot(a, b, preferred_element_type=jnp.float32,
                   precision=jax.lax.Precision.HIGHEST)


def _in_proj_body(x_ref, w_ref, b_ref, o_ref):
    y = _dot(x_ref[...], w_ref[...]) + b_ref[0]
    o_ref[0] = y[:, :HH]
    o_ref[1] = y[:, HH:]


def _layer_compute(h_ref, agg_ref, cnt_ref, wl_ref, wr_ref, wc_ref,
                   bl_ref, aux_ref):
    hA = h_ref[0]
    hB = h_ref[1]
    acc = jnp.broadcast_to(aux_ref[2], (BLK, H))
    for t in range(T):
        ar = agg_ref[t]
        cr = cnt_ref[t]
        cnt = cr[0] + cr[1]
        inv = 1.0 / jnp.maximum(cnt[:, :1], 1.0)
        wl = wl_ref[t]
        wr = wr_ref[t]
        su = _dot(ar[0], wl[:HH]) + _dot(ar[1], wl[HH:])
        su = su * inv + bl_ref[t]
        su = su + _dot(hA, wr[:HH]) + _dot(hB, wr[HH:])
        nrm = jnp.sqrt(jnp.sum(su * su, axis=1, keepdims=True))
        su = su / jnp.maximum(nrm, 1e-12)
        acc = acc + _dot(su, wc_ref[t])
    mu = jnp.mean(acc, axis=1, keepdims=True)
    var = jnp.mean((acc - mu) ** 2, axis=1, keepdims=True)
    y = (acc - mu) * lax.rsqrt(var + 1e-5) * aux_ref[0] + aux_ref[1]
    return jnp.maximum(y, 0.0)


def _layer_body(h_ref, agg_ref, cnt_ref, wl_ref, wr_ref, wc_ref,
                bl_ref, aux_ref, o_ref):
    y = _layer_compute(h_ref, agg_ref, cnt_ref, wl_ref, wr_ref, wc_ref,
                       bl_ref, aux_ref)
    o_ref[0] = y[:, :HH]
    o_ref[1] = y[:, HH:]


def _layer_final_body(h_ref, agg_ref, cnt_ref, wl_ref, wr_ref, wc_ref,
                      bl_ref, aux_ref, wo_ref, bo_ref, o_ref):
    y = _layer_compute(h_ref, agg_ref, cnt_ref, wl_ref, wr_ref, wc_ref,
                       bl_ref, aux_ref)
    o_ref[...] = _dot(y, wo_ref[...]) + bo_ref[0]


def _full_spec(shape):
    return pl.BlockSpec(shape, lambda i: tuple(0 for _ in shape))


_SPLIT_SPEC = pl.BlockSpec((NC, BLK, HH), lambda i: (0, i, 0))

_in_proj = pl.pallas_call(
    _in_proj_body,
    grid=(GRID,),
    in_specs=[
        pl.BlockSpec((BLK, H), lambda i: (i, 0)),
        _full_spec((H, H)),
        _full_spec((8, H)),
    ],
    out_specs=_SPLIT_SPEC,
    out_shape=jax.ShapeDtypeStruct((NC, N, HH), jnp.float32),
)

_layer = pl.pallas_call(
    _layer_body,
    grid=(GRID,),
    in_specs=[
        _SPLIT_SPEC,
        pl.BlockSpec((T, NC, BLK, HH), lambda i: (0, 0, i, 0)),
        pl.BlockSpec((T, NC, BLK, HH), lambda i: (0, 0, i, 0)),
        _full_spec((T, H, H)),
        _full_spec((T, H, H)),
        _full_spec((T, H, H)),
        _full_spec((8, H)),
        _full_spec((8, H)),
    ],
    out_specs=_SPLIT_SPEC,
    out_shape=jax.ShapeDtypeStruct((NC, N, HH), jnp.float32),
)

_layer_final = pl.pallas_call(
    _layer_final_body,
    grid=(GRID,),
    in_specs=[
        _SPLIT_SPEC,
        pl.BlockSpec((T, NC, BLK, HH), lambda i: (0, 0, i, 0)),
        pl.BlockSpec((T, NC, BLK, HH), lambda i: (0, 0, i, 0)),
        _full_spec((T, H, H)),
        _full_spec((T, H, H)),
        _full_spec((T, H, H)),
        _full_spec((8, H)),
        _full_spec((8, H)),
        _full_spec((H, DOUT)),
        _full_spec((8, DOUT)),
    ],
    out_specs=pl.BlockSpec((BLK, DOUT), lambda i: (i, 0)),
    out_shape=jax.ShapeDtypeStruct((N, DOUT), jnp.float32),
)


def _pad8(v2d):
    return jnp.zeros((8, v2d.shape[1]), jnp.float32).at[: v2d.shape[0]].set(
        v2d)


def kernel(x, edge_index_0, edge_index_1, edge_index_2, Win, bin_, Wl, bl, Wr,
           edge_att, Wc, bc, gamma, beta, Wout, bout):
    eis = (edge_index_0, edge_index_1, edge_index_2)

    pad_rows = (jnp.arange(max(PAD16, PAD32), dtype=jnp.int32) % 16)
    src16s, dst16s, dst32s = [], [], []
    for ei in eis:
        src = ei[0]
        dst = ei[1]
        sp = jnp.concatenate([src, pad_rows[:PAD16]])
        dp = jnp.concatenate([dst, N + pad_rows[:PAD16]])
        src16s.append(jnp.stack([sp, sp + N]).reshape(NC, NS, C16, CHUNK))
        dst16s.append(dp.reshape(NS, C16, CHUNK))
        dst32s.append(jnp.concatenate([dst, N + pad_rows[:PAD32]]).reshape(
            NC * NS, C32, CHUNK))
    src_all = jnp.stack(src16s)
    dst_all = jnp.stack(dst16s)
    dst32_all = jnp.stack(dst32s)

    zrows = jnp.zeros((ROWS_PT, HH), jnp.float32)

    ones_r = jnp.ones((CHUNK, HH), jnp.float32)
    cnt = _sc_count(dst32_all, ones_r, zrows)

    winT = Win.T
    binp = _pad8(bin_[None, :])
    wlT = jnp.transpose(Wl, (0, 1, 3, 2))
    wrT = jnp.transpose(Wr, (0, 1, 3, 2))
    wcT = jnp.transpose(Wc, (0, 2, 1)).reshape(L, T, H, H) * \
        edge_att[:, :, None, None]
    woutT = Wout.T
    boutp = _pad8(bout[None, :])

    h2 = _in_proj(x, winT, binp)
    for i in range(L):
        htab = h2.reshape(NC * N, HH)
        agg = _sc_agg(htab, src_all, dst_all, zrows, cnt)
        blp = _pad8(bl[i])
        aux = _pad8(jnp.stack([gamma[i], beta[i], bc[i]]))
        args = (h2, agg, cnt, wlT[i], wrT[i], wcT[i], blp, aux)
        if i + 1 < L:
            h2 = _layer(*args)
        else:
            return _layer_final(*args, woutT, boutp)


# speedup vs baseline: 1.2952x; 1.2952x over previous
import functools

import jax
import jax.numpy as jnp
from jax import lax
from jax.experimental import pallas as pl
from jax.experimental.pallas import tpu as pltpu
from jax.experimental.pallas import tpu_sc as plsc

N = 10000
E = 160000
H = 256
HH = 128
DOUT = 128
L = 3
T = 3

NC = 2
NS = 16
CHUNK = 128

IDXBLK = 16
C16 = 80
PAD16 = NS * C16 * CHUNK - E
C32 = -(-(E // (NC * NS)) // CHUNK)
PAD32 = NC * NS * C32 * CHUNK - E

ROWS_PT = 632
NPAD = NS * ROWS_PT
BLK = 1000
GRID = N // BLK


@functools.partial(
    pl.kernel,
    out_type=jax.ShapeDtypeStruct((T, NC, NPAD, HH), jnp.float32),
    mesh=plsc.VectorSubcoreMesh(core_axis_name="c", subcore_axis_name="s", num_cores=NC, num_subcores=NS),
    scratch_types=[
        pltpu.VMEM((2, IDXBLK, CHUNK), jnp.int32),
        pltpu.VMEM((2, IDXBLK, CHUNK), jnp.int32),
        pltpu.VMEM((2, CHUNK, HH), jnp.float32),
        pltpu.VMEM_SHARED((NPAD, HH), jnp.float32),
        pltpu.SemaphoreType.DMA,
        pltpu.SemaphoreType.DMA,
        pltpu.SemaphoreType.DMA,
    ],
)
def _sc_agg(h_hbm, src_hbm, dst_hbm, z_hbm, dep_hbm, out_hbm, sidx, didx, gbuf,
            aggsh, gsem, ssem, isem):
    c = lax.axis_index("c")
    s = lax.axis_index("s")
    r0 = s * ROWS_PT
    pltpu.sync_copy(z_hbm, aggsh.at[pl.ds(r0, ROWS_PT)])
    plsc.subcore_barrier()

    NMB = C16 // IDXBLK
    blocks = [(t, m) for t in range(T) for m in range(NMB)]

    def _stage_idx(t, m, slot, copy):
        copy(src_hbm.at[t, c, s, pl.ds(m * IDXBLK, IDXBLK)], sidx.at[slot])
        copy(dst_hbm.at[t, s, pl.ds(m * IDXBLK, IDXBLK)], didx.at[slot])

    _stage_idx(0, 0, 0, pltpu.sync_copy)
    pltpu.async_copy(h_hbm.at[sidx.at[0, 0]], gbuf.at[0], gsem)

    for k, (t, m) in enumerate(blocks):
        ks = k % 2
        nks = (k + 1) % 2
        if k + 1 < len(blocks):
            nt, nm = blocks[k + 1]
            _stage_idx(nt, nm, nks,
                       lambda a, b: pltpu.async_copy(a, b, isem))

        def body(j, carry, ks=ks):
            cur = lax.rem(j, 2)
            nxt = lax.rem(j + 1, 2)

            pltpu.make_async_copy(h_hbm.at[sidx.at[ks, j]], gbuf.at[cur],
                                  gsem).wait()
            pltpu.async_copy(gbuf.at[cur], aggsh.at[didx.at[ks, j]], ssem,
                             add=True)

            @pl.when(j >= 1)
            def _():
                pltpu.make_async_copy(z_hbm.at[pl.ds(0, CHUNK)],
                                      gbuf.at[nxt], ssem).wait()

            @pl.when(j + 1 < IDXBLK)
            def _():
                pltpu.async_copy(h_hbm.at[sidx.at[ks, j + 1]], gbuf.at[nxt],
                                 gsem)
            return carry

        lax.fori_loop(0, IDXBLK, body, 0)
        pltpu.make_async_copy(z_hbm.at[pl.ds(0, CHUNK)], gbuf.at[0],
                              ssem).wait()
        if k + 1 < len(blocks):
            nt, nm = blocks[k + 1]
            _stage_idx(nt, nm, nks,
                       lambda a, b: pltpu.make_async_copy(a, b, isem).wait())
            pltpu.async_copy(h_hbm.at[sidx.at[nks, 0]], gbuf.at[0], gsem)

        if m == NMB - 1:
            plsc.subcore_barrier()
            pltpu.sync_copy(aggsh.at[pl.ds(r0, ROWS_PT)],
                            out_hbm.at[t, c, pl.ds(r0, ROWS_PT)])
            if t + 1 < T:
                pltpu.sync_copy(z_hbm, aggsh.at[pl.ds(r0, ROWS_PT)])
                plsc.subcore_barrier()


@functools.partial(
    pl.kernel,
    out_type=jax.ShapeDtypeStruct((T, NC, NPAD, HH), jnp.float32),
    mesh=plsc.VectorSubcoreMesh(core_axis_name="c", subcore_axis_name="s", num_cores=NC, num_subcores=NS),
    scratch_types=[
        pltpu.VMEM((C32, CHUNK), jnp.int32),
        pltpu.VMEM((CHUNK, HH), jnp.float32),
        pltpu.VMEM_SHARED((NPAD, HH), jnp.float32),
        pltpu.SemaphoreType.DMA,
    ],
)
def _sc_count(dst_hbm, ones_hbm, z_hbm, out_hbm, didx, ones, cntsh, ssem):
    c = lax.axis_index("c")
    s = lax.axis_index("s")
    wid = s * NC + c
    r0 = s * ROWS_PT
    pltpu.sync_copy(ones_hbm, ones)
    pltpu.sync_copy(z_hbm, cntsh.at[pl.ds(r0, ROWS_PT)])
    plsc.subcore_barrier()

    for t in range(T):
        pltpu.sync_copy(dst_hbm.at[t, wid], didx)

        def body(j, carry):
            pltpu.async_copy(ones, cntsh.at[didx.at[j]], ssem, add=True)
            return carry

        lax.fori_loop(0, C32, body, 0)

        def drain(j, carry):
            pltpu.make_async_copy(z_hbm.at[pl.ds(0, CHUNK)], ones,
                                  ssem).wait()
            return carry

        lax.fori_loop(0, C32, drain, 0)
        plsc.subcore_barrier()
        pltpu.sync_copy(cntsh.at[pl.ds(r0, ROWS_PT)],
                        out_hbm.at[t, c, pl.ds(r0, ROWS_PT)])
        if t + 1 < T:
            pltpu.sync_copy(z_hbm, cntsh.at[pl.ds(r0, ROWS_PT)])
            plsc.subcore_barrier()


def _dot(a, b):
    return jnp.dot(a, b, preferred_element_type=jnp.float32)


def _in_proj_body(x_ref, w_ref, b_ref, o_ref):
    y = _dot(x_ref[...], w_ref[...]) + b_ref[0]
    o_ref[0] = y[:, :HH]
    o_ref[1] = y[:, HH:]


def _layer_compute(h_ref, agg_ref, cnt_ref, wl_ref, wr_ref, wc_ref,
                   bl_ref, aux_ref):
    hA = h_ref[0]
    hB = h_ref[1]
    acc = jnp.broadcast_to(aux_ref[2], (BLK, H))
    for t in range(T):
        ar = agg_ref[t]
        cr = cnt_ref[t]
        cnt = cr[0] + cr[1]
        inv = 1.0 / jnp.maximum(cnt[:, :1], 1.0)
        wl = wl_ref[t]
        wr = wr_ref[t]
        su = _dot(ar[0], wl[:HH]) + _dot(ar[1], wl[HH:])
        su = su * inv + bl_ref[t]
        su = su + _dot(hA, wr[:HH]) + _dot(hB, wr[HH:])
        nrm = jnp.sqrt(jnp.sum(su * su, axis=1, keepdims=True))
        su = su / jnp.maximum(nrm, 1e-12)
        acc = acc + _dot(su, wc_ref[t])
    mu = jnp.mean(acc, axis=1, keepdims=True)
    var = jnp.mean((acc - mu) ** 2, axis=1, keepdims=True)
    y = (acc - mu) * lax.rsqrt(var + 1e-5) * aux_ref[0] + aux_ref[1]
    return jnp.maximum(y, 0.0)


def _layer_body(h_ref, agg_ref, cnt_ref, wl_ref, wr_ref, wc_ref,
                bl_ref, aux_ref, o_ref):
    y = _layer_compute(h_ref, agg_ref, cnt_ref, wl_ref, wr_ref, wc_ref,
                       bl_ref, aux_ref)
    o_ref[0] = y[:, :HH]
    o_ref[1] = y[:, HH:]


def _layer_final_body(h_ref, agg_ref, cnt_ref, wl_ref, wr_ref, wc_ref,
                      bl_ref, aux_ref, wo_ref, bo_ref, o_ref):
    y = _layer_compute(h_ref, agg_ref, cnt_ref, wl_ref, wr_ref, wc_ref,
                       bl_ref, aux_ref)
    o_ref[...] = _dot(y, wo_ref[...]) + bo_ref[0]


def _full_spec(shape):
    return pl.BlockSpec(shape, lambda i: tuple(0 for _ in shape))


_SPLIT_SPEC = pl.BlockSpec((NC, BLK, HH), lambda i: (0, i, 0))

_in_proj = pl.pallas_call(
    _in_proj_body,
    grid=(GRID,),
    in_specs=[
        pl.BlockSpec((BLK, H), lambda i: (i, 0)),
        _full_spec((H, H)),
        _full_spec((8, H)),
    ],
    out_specs=_SPLIT_SPEC,
    out_shape=jax.ShapeDtypeStruct((NC, N, HH), jnp.float32),
)

_layer = pl.pallas_call(
    _layer_body,
    grid=(GRID,),
    in_specs=[
        _SPLIT_SPEC,
        pl.BlockSpec((T, NC, BLK, HH), lambda i: (0, 0, i, 0)),
        pl.BlockSpec((T, NC, BLK, HH), lambda i: (0, 0, i, 0)),
        _full_spec((T, H, H)),
        _full_spec((T, H, H)),
        _full_spec((T, H, H)),
        _full_spec((8, H)),
        _full_spec((8, H)),
    ],
    out_specs=_SPLIT_SPEC,
    out_shape=jax.ShapeDtypeStruct((NC, N, HH), jnp.float32),
)

_layer_final = pl.pallas_call(
    _layer_final_body,
    grid=(GRID,),
    in_specs=[
        _SPLIT_SPEC,
        pl.BlockSpec((T, NC, BLK, HH), lambda i: (0, 0, i, 0)),
        pl.BlockSpec((T, NC, BLK, HH), lambda i: (0, 0, i, 0)),
        _full_spec((T, H, H)),
        _full_spec((T, H, H)),
        _full_spec((T, H, H)),
        _full_spec((8, H)),
        _full_spec((8, H)),
        _full_spec((H, DOUT)),
        _full_spec((8, DOUT)),
    ],
    out_specs=pl.BlockSpec((BLK, DOUT), lambda i: (i, 0)),
    out_shape=jax.ShapeDtypeStruct((N, DOUT), jnp.float32),
)


def _pad8(v2d):
    return jnp.zeros((8, v2d.shape[1]), jnp.float32).at[: v2d.shape[0]].set(
        v2d)


def kernel(x, edge_index_0, edge_index_1, edge_index_2, Win, bin_, Wl, bl, Wr,
           edge_att, Wc, bc, gamma, beta, Wout, bout):
    eis = (edge_index_0, edge_index_1, edge_index_2)

    pad_rows = (jnp.arange(max(PAD16, PAD32), dtype=jnp.int32) % 16)
    src16s, dst16s, dst32s = [], [], []
    for ei in eis:
        src = ei[0]
        dst = ei[1]
        sp = jnp.concatenate([src, pad_rows[:PAD16]])
        dp = jnp.concatenate([dst, N + pad_rows[:PAD16]])
        src16s.append(jnp.stack([sp, sp + N]).reshape(NC, NS, C16, CHUNK))
        dst16s.append(dp.reshape(NS, C16, CHUNK))
        dst32s.append(jnp.concatenate([dst, N + pad_rows[:PAD32]]).reshape(
            NC * NS, C32, CHUNK))
    src_all = jnp.stack(src16s)
    dst_all = jnp.stack(dst16s)
    dst32_all = jnp.stack(dst32s)

    zrows = jnp.zeros((ROWS_PT, HH), jnp.float32)

    ones_r = jnp.ones((CHUNK, HH), jnp.float32)
    cnt = _sc_count(dst32_all, ones_r, zrows)

    winT = Win.T
    binp = _pad8(bin_[None, :])
    wlT = jnp.transpose(Wl, (0, 1, 3, 2))
    wrT = jnp.transpose(Wr, (0, 1, 3, 2))
    wcT = jnp.transpose(Wc, (0, 2, 1)).reshape(L, T, H, H) * \
        edge_att[:, :, None, None]
    woutT = Wout.T
    boutp = _pad8(bout[None, :])

    h2 = _in_proj(x, winT, binp)
    for i in range(L):
        htab = h2.reshape(NC * N, HH)
        agg = _sc_agg(htab, src_all, dst_all, zrows, cnt)
        blp = _pad8(bl[i])
        aux = _pad8(jnp.stack([gamma[i], beta[i], bc[i]]))
        args = (h2, agg, cnt, wlT[i], wrT[i], wcT[i], blp, aux)
        if i + 1 < L:
            h2 = _layer(*args)
        else:
            return _layer_final(*args, woutT, boutp)


# speedup vs baseline: 1.3062x; 1.0085x over previous
import functools

import jax
import jax.numpy as jnp
from jax import lax
from jax.experimental import pallas as pl
from jax.experimental.pallas import tpu as pltpu
from jax.experimental.pallas import tpu_sc as plsc

N = 10000
E = 160000
H = 256
HH = 128
DOUT = 128
L = 3
T = 3

NC = 2
NS = 16
CHUNK = 128

IDXBLK = 16
C16 = 80
PAD16 = NS * C16 * CHUNK - E
C32 = -(-(E // (NC * NS)) // CHUNK)
PAD32 = NC * NS * C32 * CHUNK - E

ROWS_PT = 632
NPAD = NS * ROWS_PT
BLK = 2000
GRID = N // BLK


@functools.partial(
    pl.kernel,
    out_type=jax.ShapeDtypeStruct((T, NC, NPAD, HH), jnp.float32),
    mesh=plsc.VectorSubcoreMesh(core_axis_name="c", subcore_axis_name="s", num_cores=NC, num_subcores=NS),
    scratch_types=[
        pltpu.VMEM((2, IDXBLK, CHUNK), jnp.int32),
        pltpu.VMEM((2, IDXBLK, CHUNK), jnp.int32),
        pltpu.VMEM((2, CHUNK, HH), jnp.float32),
        pltpu.VMEM_SHARED((NPAD, HH), jnp.float32),
        pltpu.SemaphoreType.DMA,
        pltpu.SemaphoreType.DMA,
        pltpu.SemaphoreType.DMA,
    ],
)
def _sc_agg(h_hbm, src_hbm, dst_hbm, z_hbm, dep_hbm, out_hbm, sidx, didx, gbuf,
            aggsh, gsem, ssem, isem):
    c = lax.axis_index("c")
    s = lax.axis_index("s")
    r0 = s * ROWS_PT
    pltpu.sync_copy(z_hbm, aggsh.at[pl.ds(r0, ROWS_PT)])
    plsc.subcore_barrier()

    NMB = C16 // IDXBLK
    blocks = [(t, m) for t in range(T) for m in range(NMB)]

    def _stage_idx(t, m, slot, copy):
        copy(src_hbm.at[t, c, s, pl.ds(m * IDXBLK, IDXBLK)], sidx.at[slot])
        copy(dst_hbm.at[t, s, pl.ds(m * IDXBLK, IDXBLK)], didx.at[slot])

    _stage_idx(0, 0, 0, pltpu.sync_copy)
    pltpu.async_copy(h_hbm.at[sidx.at[0, 0]], gbuf.at[0], gsem)

    for k, (t, m) in enumerate(blocks):
        ks = k % 2
        nks = (k + 1) % 2
        if k + 1 < len(blocks):
            nt, nm = blocks[k + 1]
            _stage_idx(nt, nm, nks,
                       lambda a, b: pltpu.async_copy(a, b, isem))

        def body(j, carry, ks=ks):
            cur = lax.rem(j, 2)
            nxt = lax.rem(j + 1, 2)

            pltpu.make_async_copy(h_hbm.at[sidx.at[ks, j]], gbuf.at[cur],
                                  gsem).wait()
            pltpu.async_copy(gbuf.at[cur], aggsh.at[didx.at[ks, j]], ssem,
                             add=True)

            @pl.when(j >= 1)
            def _():
                pltpu.make_async_copy(z_hbm.at[pl.ds(0, CHUNK)],
                                      gbuf.at[nxt], ssem).wait()

            @pl.when(j + 1 < IDXBLK)
            def _():
                pltpu.async_copy(h_hbm.at[sidx.at[ks, j + 1]], gbuf.at[nxt],
                                 gsem)
            return carry

        lax.fori_loop(0, IDXBLK, body, 0)
        pltpu.make_async_copy(z_hbm.at[pl.ds(0, CHUNK)], gbuf.at[0],
                              ssem).wait()
        if k + 1 < len(blocks):
            nt, nm = blocks[k + 1]
            _stage_idx(nt, nm, nks,
                       lambda a, b: pltpu.make_async_copy(a, b, isem).wait())
            pltpu.async_copy(h_hbm.at[sidx.at[nks, 0]], gbuf.at[0], gsem)

        if m == NMB - 1:
            plsc.subcore_barrier()
            pltpu.sync_copy(aggsh.at[pl.ds(r0, ROWS_PT)],
                            out_hbm.at[t, c, pl.ds(r0, ROWS_PT)])
            if t + 1 < T:
                pltpu.sync_copy(z_hbm, aggsh.at[pl.ds(r0, ROWS_PT)])
                plsc.subcore_barrier()


@functools.partial(
    pl.kernel,
    out_type=jax.ShapeDtypeStruct((T, NC, NPAD, HH), jnp.float32),
    mesh=plsc.VectorSubcoreMesh(core_axis_name="c", subcore_axis_name="s", num_cores=NC, num_subcores=NS),
    scratch_types=[
        pltpu.VMEM((C32, CHUNK), jnp.int32),
        pltpu.VMEM((CHUNK, HH), jnp.float32),
        pltpu.VMEM_SHARED((NPAD, HH), jnp.float32),
        pltpu.SemaphoreType.DMA,
    ],
)
def _sc_count(dst_hbm, ones_hbm, z_hbm, out_hbm, didx, ones, cntsh, ssem):
    c = lax.axis_index("c")
    s = lax.axis_index("s")
    wid = s * NC + c
    r0 = s * ROWS_PT
    pltpu.sync_copy(ones_hbm, ones)
    pltpu.sync_copy(z_hbm, cntsh.at[pl.ds(r0, ROWS_PT)])
    plsc.subcore_barrier()

    for t in range(T):
        pltpu.sync_copy(dst_hbm.at[t, wid], didx)

        def body(j, carry):
            pltpu.async_copy(ones, cntsh.at[didx.at[j]], ssem, add=True)
            return carry

        lax.fori_loop(0, C32, body, 0)

        def drain(j, carry):
            pltpu.make_async_copy(z_hbm.at[pl.ds(0, CHUNK)], ones,
                                  ssem).wait()
            return carry

        lax.fori_loop(0, C32, drain, 0)
        plsc.subcore_barrier()
        pltpu.sync_copy(cntsh.at[pl.ds(r0, ROWS_PT)],
                        out_hbm.at[t, c, pl.ds(r0, ROWS_PT)])
        if t + 1 < T:
            pltpu.sync_copy(z_hbm, cntsh.at[pl.ds(r0, ROWS_PT)])
            plsc.subcore_barrier()


def _dot(a, b):
    return jnp.dot(a, b, preferred_element_type=jnp.float32)


def _in_proj_body(x_ref, w_ref, b_ref, o_ref):
    y = _dot(x_ref[...], w_ref[...]) + b_ref[0]
    o_ref[0] = y[:, :HH]
    o_ref[1] = y[:, HH:]


def _layer_compute(h_ref, agg_ref, cnt_ref, wl_ref, wr_ref, wc_ref,
                   bl_ref, aux_ref):
    hA = h_ref[0]
    hB = h_ref[1]
    acc = jnp.broadcast_to(aux_ref[2], (BLK, H))
    for t in range(T):
        ar = agg_ref[t]
        cr = cnt_ref[t]
        cnt = cr[0] + cr[1]
        inv = 1.0 / jnp.maximum(cnt[:, :1], 1.0)
        wl = wl_ref[t]
        wr = wr_ref[t]
        su = _dot(ar[0], wl[:HH]) + _dot(ar[1], wl[HH:])
        su = su * inv + bl_ref[t]
        su = su + _dot(hA, wr[:HH]) + _dot(hB, wr[HH:])
        nrm = jnp.sqrt(jnp.sum(su * su, axis=1, keepdims=True))
        su = su / jnp.maximum(nrm, 1e-12)
        acc = acc + _dot(su, wc_ref[t])
    mu = jnp.mean(acc, axis=1, keepdims=True)
    var = jnp.mean((acc - mu) ** 2, axis=1, keepdims=True)
    y = (acc - mu) * lax.rsqrt(var + 1e-5) * aux_ref[0] + aux_ref[1]
    return jnp.maximum(y, 0.0)


def _layer_body(h_ref, agg_ref, cnt_ref, wl_ref, wr_ref, wc_ref,
                bl_ref, aux_ref, o_ref):
    y = _layer_compute(h_ref, agg_ref, cnt_ref, wl_ref, wr_ref, wc_ref,
                       bl_ref, aux_ref)
    o_ref[0] = y[:, :HH]
    o_ref[1] = y[:, HH:]


def _layer_final_body(h_ref, agg_ref, cnt_ref, wl_ref, wr_ref, wc_ref,
                      bl_ref, aux_ref, wo_ref, bo_ref, o_ref):
    y = _layer_compute(h_ref, agg_ref, cnt_ref, wl_ref, wr_ref, wc_ref,
                       bl_ref, aux_ref)
    o_ref[...] = _dot(y, wo_ref[...]) + bo_ref[0]


def _full_spec(shape):
    return pl.BlockSpec(shape, lambda i: tuple(0 for _ in shape))


_SPLIT_SPEC = pl.BlockSpec((NC, BLK, HH), lambda i: (0, i, 0))

_in_proj = pl.pallas_call(
    _in_proj_body,
    grid=(GRID,),
    in_specs=[
        pl.BlockSpec((BLK, H), lambda i: (i, 0)),
        _full_spec((H, H)),
        _full_spec((8, H)),
    ],
    out_specs=_SPLIT_SPEC,
    out_shape=jax.ShapeDtypeStruct((NC, N, HH), jnp.float32),
)

_layer = pl.pallas_call(
    _layer_body,
    grid=(GRID,),
    in_specs=[
        _SPLIT_SPEC,
        pl.BlockSpec((T, NC, BLK, HH), lambda i: (0, 0, i, 0)),
        pl.BlockSpec((T, NC, BLK, HH), lambda i: (0, 0, i, 0)),
        _full_spec((T, H, H)),
        _full_spec((T, H, H)),
        _full_spec((T, H, H)),
        _full_spec((8, H)),
        _full_spec((8, H)),
    ],
    out_specs=_SPLIT_SPEC,
    out_shape=jax.ShapeDtypeStruct((NC, N, HH), jnp.float32),
)

_layer_final = pl.pallas_call(
    _layer_final_body,
    grid=(GRID,),
    in_specs=[
        _SPLIT_SPEC,
        pl.BlockSpec((T, NC, BLK, HH), lambda i: (0, 0, i, 0)),
        pl.BlockSpec((T, NC, BLK, HH), lambda i: (0, 0, i, 0)),
        _full_spec((T, H, H)),
        _full_spec((T, H, H)),
        _full_spec((T, H, H)),
        _full_spec((8, H)),
        _full_spec((8, H)),
        _full_spec((H, DOUT)),
        _full_spec((8, DOUT)),
    ],
    out_specs=pl.BlockSpec((BLK, DOUT), lambda i: (i, 0)),
    out_shape=jax.ShapeDtypeStruct((N, DOUT), jnp.float32),
)


def _pad8(v2d):
    return jnp.zeros((8, v2d.shape[1]), jnp.float32).at[: v2d.shape[0]].set(
        v2d)


def kernel(x, edge_index_0, edge_index_1, edge_index_2, Win, bin_, Wl, bl, Wr,
           edge_att, Wc, bc, gamma, beta, Wout, bout):
    eis = (edge_index_0, edge_index_1, edge_index_2)

    pad_rows = (jnp.arange(max(PAD16, PAD32), dtype=jnp.int32) % 16)
    src16s, dst16s, dst32s = [], [], []
    for ei in eis:
        src = ei[0]
        dst = ei[1]
        sp = jnp.concatenate([src, pad_rows[:PAD16]])
        dp = jnp.concatenate([dst, N + pad_rows[:PAD16]])
        src16s.append(jnp.stack([sp, sp + N]).reshape(NC, NS, C16, CHUNK))
        dst16s.append(dp.reshape(NS, C16, CHUNK))
        dst32s.append(jnp.concatenate([dst, N + pad_rows[:PAD32]]).reshape(
            NC * NS, C32, CHUNK))
    src_all = jnp.stack(src16s)
    dst_all = jnp.stack(dst16s)
    dst32_all = jnp.stack(dst32s)

    zrows = jnp.zeros((ROWS_PT, HH), jnp.float32)

    ones_r = jnp.ones((CHUNK, HH), jnp.float32)
    cnt = _sc_count(dst32_all, ones_r, zrows)

    winT = Win.T
    binp = _pad8(bin_[None, :])
    wlT = jnp.transpose(Wl, (0, 1, 3, 2))
    wrT = jnp.transpose(Wr, (0, 1, 3, 2))
    wcT = jnp.transpose(Wc, (0, 2, 1)).reshape(L, T, H, H) * \
        edge_att[:, :, None, None]
    woutT = Wout.T
    boutp = _pad8(bout[None, :])

    h2 = _in_proj(x, winT, binp)
    for i in range(L):
        htab = h2.reshape(NC * N, HH)
        agg = _sc_agg(htab, src_all, dst_all, zrows, cnt)
        blp = _pad8(bl[i])
        aux = _pad8(jnp.stack([gamma[i], beta[i], bc[i]]))
        args = (h2, agg, cnt, wlT[i], wrT[i], wcT[i], blp, aux)
        if i + 1 < L:
            h2 = _layer(*args)
        else:
            return _layer_final(*args, woutT, boutp)
